# baseline (speedup 1.0000x reference)
import jax
import jax.numpy as jnp
from jax.experimental import pallas as pl
from jax.experimental.pallas import tpu as pltpu

_N = 8192
_K = 8
_R = 256


def _knn_tile(xq_ref, xt_ref, y_ref, out_ref, d_ref):
    t = pl.program_id(0)
    xq = xq_ref[...]
    xt = xt_ref[...]
    y = y_ref[...]

    g = jax.lax.dot_general(
        xq, xt, (((1,), (0,)), ((), ())),
        preferred_element_type=jnp.float32)
    sq_q = xq[:, 0:1] * xq[:, 0:1] + xq[:, 1:2] * xq[:, 1:2]
    sq_a = xt[0:1, :] * xt[0:1, :] + xt[1:2, :] * xt[1:2, :]
    d = (sq_q + sq_a) - 2.0 * g
    d = jnp.maximum(d, 0.0)

    rows = t * _R + jax.lax.broadcasted_iota(jnp.int32, (_R, 1), 0)
    cols = jax.lax.broadcasted_iota(jnp.int32, (_R, _N), 1)
    d = jnp.where(cols == rows, jnp.inf, d)
    d_ref[...] = d

    s = jnp.zeros((_R, 1), jnp.float32)
    m = None
    jm = None
    for p in range(_K + 1):
        d = d_ref[...]
        if p > 0:
            hit = cols == jm
            d = jnp.where(hit, jnp.inf, d)
            d_ref[...] = d
            lab = jnp.sum(jnp.where(hit, y, 0.0), axis=1, keepdims=True)
            s = s + jnp.where(m < 999.0, lab, 0.0)
        if p < _K:
            m = jnp.min(d, axis=1, keepdims=True)
            jm = jnp.min(jnp.where(d == m, cols, _N), axis=1, keepdims=True)

    out_ref[...] = (s > (_K / 2.0)).astype(jnp.float32)


def kernel(x, y):
    n = x.shape[0]
    yf = y.astype(jnp.float32).reshape(1, n)
    xt = x.T

    out = pl.pallas_call(
        _knn_tile,
        grid=(n // _R,),
        in_specs=[
            pl.BlockSpec((_R, 2), lambda i: (i, 0)),
            pl.BlockSpec((2, n), lambda i: (0, 0)),
            pl.BlockSpec((1, n), lambda i: (0, 0)),
        ],
        out_specs=pl.BlockSpec((_R, 1), lambda i: (i, 0)),
        out_shape=jax.ShapeDtypeStruct((n, 1), jnp.float32),
        scratch_shapes=[pltpu.VMEM((_R, _N), jnp.float32)],
    )(x, xt, yf)
    return out.reshape(n)


# speedup vs baseline: 1.0014x; 1.0014x over previous
import functools

import jax
import jax.numpy as jnp
from jax import lax
from jax.experimental import pallas as pl
from jax.experimental.pallas import tpu as pltpu
from jax.experimental.pallas import tpu_sc as plsc

_N = 8192
_K = 8
_R = 256
_C = 64
_CW = _N // _C
_NSUB = 32
_RPS = _N // _NSUB
_B = 16
_NB = _RPS // _B
_CAND = 1088


def _tc_tile(xq_ref, xt_ref, out_d_ref, c8_ref, tau_ref):
    t = pl.program_id(0)
    xq = xq_ref[...]
    xt = xt_ref[...]

    g = jax.lax.dot_general(
        xq, xt, (((1,), (0,)), ((), ())),
        preferred_element_type=jnp.float32)
    sq_q = xq[:, 0:1] * xq[:, 0:1] + xq[:, 1:2] * xq[:, 1:2]
    sq_a = xt[0:1, :] * xt[0:1, :] + xt[1:2, :] * xt[1:2, :]
    d = (sq_q + sq_a) - 2.0 * g
    d = jnp.maximum(d, 0.0)

    rows = t * _R + jax.lax.broadcasted_iota(jnp.int32, (_R, 1), 0)
    cols = jax.lax.broadcasted_iota(jnp.int32, (_R, _N), 1)
    d = jnp.where(cols == rows, jnp.inf, d)
    out_d_ref[...] = d

    mins = [
        jnp.min(d[:, c * _CW:(c + 1) * _CW], axis=1, keepdims=True)
        for c in range(_C)
    ]
    m = jnp.concatenate(mins, axis=1)

    ciota = jax.lax.broadcasted_iota(jnp.int32, (_R, _C), 1)
    picked = jnp.zeros((_R, _C), jnp.bool_)
    tau = jnp.zeros((_R, 1), jnp.float32)
    for _ in range(_K):
        mv = jnp.min(m, axis=1, keepdims=True)
        cm = jnp.min(jnp.where(m == mv, ciota, _C), axis=1, keepdims=True)
        hit = ciota == cm
        picked = picked | hit
        m = jnp.where(hit, jnp.inf, m)
        tau = mv
    ids = []
    pick = picked
    for _ in range(_K):
        cq = jnp.min(jnp.where(pick, ciota, _C), axis=1, keepdims=True)
        pick = pick & (ciota != cq)
        ids.append(cq)
    c8_ref[...] = jnp.concatenate(ids, axis=1)
    tau_ref[...] = tau


def _tc_phase(x):
    n = x.shape[0]
    xt = x.T
    return pl.pallas_call(
        _tc_tile,
        grid=(n // _R,),
        in_specs=[
            pl.BlockSpec((_R, 2), lambda i: (i, 0)),
            pl.BlockSpec((2, n), lambda i: (0, 0)),
        ],
        out_specs=[
            pl.BlockSpec((_R, n), lambda i: (i, 0)),
            pl.BlockSpec((_R, _K), lambda i: (i, 0)),
            pl.BlockSpec((_R, 1), lambda i: (i, 0)),
        ],
        out_shape=[
            jax.ShapeDtypeStruct((n, n), jnp.float32),
            jax.ShapeDtypeStruct((n, _K), jnp.int32),
            jax.ShapeDtypeStruct((n, 1), jnp.float32),
        ],
    )(x, xt)


def _sc_body(d_hbm, c8_hbm, tau_hbm, y_hbm, out_hbm,
             y_v, c8_v, tau_v, idx_v, jb_v, dst_v, cd_v, cj_v, out_v, sem):
    wid = lax.axis_index("s") * 2 + lax.axis_index("c")
    row0 = wid * _RPS

    pltpu.sync_copy(y_hbm, y_v)
    pltpu.sync_copy(c8_hbm.at[pl.ds(row0 * _K, _RPS * _K)],
                    c8_v.at[pl.ds(0, _RPS * _K)])
    pltpu.sync_copy(tau_hbm.at[pl.ds(row0, _RPS)], tau_v.at[pl.ds(0, _RPS)])

    iota = lax.iota(jnp.int32, 16)
    pat2 = iota // 8
    inf16 = jnp.full((16,), jnp.inf, jnp.float32)
    bigj16 = jnp.full((16,), _N, jnp.int32)

    def batch_body(b, _):
        def mk_idx(sub, _2):
            c8s = c8_v[pl.ds(b * (_B * _K) + sub * 16, 16)]
            rowv = row0 + b * _B + sub * 2 + pat2
            idx_v[pl.ds(sub * 16, 16)] = rowv * _C + c8s
            jb_v[pl.ds(sub * 16, 16)] = c8s * _CW
            return 0
        lax.fori_loop(0, 8, mk_idx, 0)
        pltpu.async_copy(d_hbm.at[idx_v], dst_v, sem).wait()

        def row_body(i, outacc):
            tau_s = tau_v[pl.ds(b * _B + i, 16)][0]

            def filt(t, n):
                g = i * _K + t // 8
                jb = jb_v[pl.ds(g, 16)][0]
                v = dst_v[g, pl.ds((t % 8) * 16, 16)]
                jvec = jb + (t % 8) * 16 + iota
                mask = v <= tau_s
                plsc.store_compressed(cd_v.at[pl.ds(n, 16)], v, mask=mask)
                plsc.store_compressed(cj_v.at[pl.ds(n, 16)], jvec, mask=mask)
                return n + plsc.all_reduce_population_count(mask)[0]

            n = lax.fori_loop(0, _C, filt, jnp.int32(0))
            cd_v[pl.ds(n, 16)] = inf16
            cj_v[pl.ds(n, 16)] = bigj16

            bd = cd_v[pl.ds(0, 16)]
            bj = cj_v[pl.ds(0, 16)]
            bd, bj = plsc.sort_key_val(bd, bj)

            def merge(gi, carry):
                bd, bj = carry
                gd = cd_v[pl.ds(gi * 16, 16)]
                gj = cj_v[pl.ds(gi * 16, 16)]
                gd, gj = plsc.sort_key_val(gd, gj)
                gdr = lax.rev(gd, (0,))
                gjr = lax.rev(gj, (0,))
                take = (bd < gdr) | ((bd == gdr) & (bj < gjr))
                md = jnp.where(take, bd, gdr)
                mj = jnp.where(take, bj, gjr)
                mjs, mds = plsc.sort_key_val(mj, md)
                bd2, bj2 = plsc.sort_key_val(mds, mjs)
                return (bd2, bj2)

            nblk = (n + 15) // 16
            bd, bj = lax.fori_loop(1, nblk, merge, (bd, bj))

            lab = plsc.load_gather(y_v, [jnp.minimum(bj, _N - 1)])
            valid = (iota < _K) & (bd < 999.0)
            s = jnp.sum(jnp.where(valid, lab, 0.0))
            res = jnp.where(s > (_K / 2.0), 1.0, 0.0)
            return jnp.where(iota == i, res, outacc)

        acc = lax.fori_loop(0, _B, row_body, jnp.zeros((16,), jnp.float32))
        out_v[pl.ds(b * _B, 16)] = acc
        return 0

    lax.fori_loop(0, _NB, batch_body, 0)
    pltpu.sync_copy(out_v.at[pl.ds(0, _RPS)], out_hbm.at[pl.ds(row0, _RPS)])


@functools.partial(
    pl.kernel,
    out_type=jax.ShapeDtypeStruct((_N,), jnp.float32),
    mesh=plsc.VectorSubcoreMesh(core_axis_name="c", subcore_axis_name="s"),
    compiler_params=pltpu.CompilerParams(needs_layout_passes=False),
    scratch_types=[
        pltpu.VMEM((_N,), jnp.float32),
        pltpu.VMEM((_RPS * _K + 16,), jnp.int32),
        pltpu.VMEM((_RPS + 16,), jnp.float32),
        pltpu.VMEM((_B * _K,), jnp.int32),
        pltpu.VMEM((_B * _K + 16,), jnp.int32),
        pltpu.VMEM((_B * _K, _CW), jnp.float32),
        pltpu.VMEM((_CAND,), jnp.float32),
        pltpu.VMEM((_CAND,), jnp.int32),
        pltpu.VMEM((_RPS,), jnp.float32),
        pltpu.SemaphoreType.DMA,
    ],
)
def _sc_phase(d_hbm, c8_hbm, tau_hbm, y_hbm, out_hbm,
              y_v, c8_v, tau_v, idx_v, jb_v, dst_v, cd_v, cj_v, out_v, sem):
    _sc_body(d_hbm, c8_hbm, tau_hbm, y_hbm, out_hbm,
             y_v, c8_v, tau_v, idx_v, jb_v, dst_v, cd_v, cj_v, out_v, sem)


def kernel(x, y):
    n = x.shape[0]
    yf = y.astype(jnp.float32)
    d, c8, tau = _tc_phase(x)
    out = _sc_phase(
        d.reshape(n * _C, _CW),
        c8.reshape(n * _K),
        tau.reshape(n),
        yf,
    )
    return out


# speedup vs baseline: 1.1886x; 1.1869x over previous
import functools

import jax
import jax.numpy as jnp
from jax import lax
from jax.experimental import pallas as pl
from jax.experimental.pallas import tpu as pltpu
from jax.experimental.pallas import tpu_sc as plsc

_N = 8192
_K = 8
_R = 256
_C = 64
_CW = _N // _C
_NSUB = 32
_RPS = _N // _NSUB
_B = 16
_NB = _RPS // _B
_CAND = 1088


def _tc_tile(xq_ref, xt_ref, out_d_ref, c8_ref, tau_ref):
    t = pl.program_id(0)
    xq = xq_ref[...]
    xt = xt_ref[...]

    g = jax.lax.dot_general(
        xq, xt, (((1,), (0,)), ((), ())),
        preferred_element_type=jnp.float32)
    sq_q = xq[:, 0:1] * xq[:, 0:1] + xq[:, 1:2] * xq[:, 1:2]
    sq_a = xt[0:1, :] * xt[0:1, :] + xt[1:2, :] * xt[1:2, :]
    d = (sq_q + sq_a) - 2.0 * g
    d = jnp.maximum(d, 0.0)

    rows = t * _R + jax.lax.broadcasted_iota(jnp.int32, (_R, 1), 0)
    cols = jax.lax.broadcasted_iota(jnp.int32, (_R, _N), 1)
    d = jnp.where(cols == rows, jnp.inf, d)
    out_d_ref[...] = d

    mins = [
        jnp.min(d[:, c * _CW:(c + 1) * _CW], axis=1, keepdims=True)
        for c in range(_C)
    ]
    m = jnp.concatenate(mins, axis=1)

    ciota = jax.lax.broadcasted_iota(jnp.int32, (_R, _C), 1)
    picked = jnp.zeros((_R, _C), jnp.bool_)
    tau = jnp.zeros((_R, 1), jnp.float32)
    for _ in range(_K):
        mv = jnp.min(m, axis=1, keepdims=True)
        cm = jnp.min(jnp.where(m == mv, ciota, _C), axis=1, keepdims=True)
        hit = ciota == cm
        picked = picked | hit
        m = jnp.where(hit, jnp.inf, m)
        tau = mv
    ids = []
    pick = picked
    for _ in range(_K):
        cq = jnp.min(jnp.where(pick, ciota, _C), axis=1, keepdims=True)
        pick = pick & (ciota != cq)
        ids.append(cq)
    c8_ref[...] = jnp.concatenate(ids, axis=1)
    tau_ref[...] = tau


def _tc_phase(x):
    n = x.shape[0]
    xt = x.T
    return pl.pallas_call(
        _tc_tile,
        grid=(n // _R,),
        in_specs=[
            pl.BlockSpec((_R, 2), lambda i: (i, 0)),
            pl.BlockSpec((2, n), lambda i: (0, 0)),
        ],
        out_specs=[
            pl.BlockSpec((_R, n), lambda i: (i, 0)),
            pl.BlockSpec((_R, _K), lambda i: (i, 0)),
            pl.BlockSpec((_R, 1), lambda i: (i, 0)),
        ],
        out_shape=[
            jax.ShapeDtypeStruct((n, n), jnp.float32),
            jax.ShapeDtypeStruct((n, _K), jnp.int32),
            jax.ShapeDtypeStruct((n, 1), jnp.float32),
        ],
    )(x, xt)


def _sc_body(d_hbm, c8_hbm, tau_hbm, y_hbm, out_hbm,
             y_v, c8_v, tau_v, idx_v, jb_v, dst_v, cd_v, cj_v, out_v, sem):
    wid = lax.axis_index("s") * 2 + lax.axis_index("c")
    row0 = wid * _RPS

    pltpu.sync_copy(y_hbm, y_v)
    pltpu.sync_copy(c8_hbm.at[pl.ds(row0 * _K, _RPS * _K)],
                    c8_v.at[pl.ds(0, _RPS * _K)])
    pltpu.sync_copy(tau_hbm.at[pl.ds(row0, _RPS)], tau_v.at[pl.ds(0, _RPS)])

    iota = lax.iota(jnp.int32, 16)
    pat2 = iota // 8
    inf16 = jnp.full((16,), jnp.inf, jnp.float32)
    bigj16 = jnp.full((16,), _N, jnp.int32)

    def batch_body(b, _):
        def mk_idx(sub, _2):
            c8s = c8_v[pl.ds(b * (_B * _K) + sub * 16, 16)]
            rowv = row0 + b * _B + sub * 2 + pat2
            idx_v[pl.ds(sub * 16, 16)] = rowv * _C + c8s
            jb_v[pl.ds(sub * 16, 16)] = c8s * _CW
            return 0
        lax.fori_loop(0, 8, mk_idx, 0)
        pltpu.async_copy(d_hbm.at[idx_v], dst_v, sem).wait()

        def row_body(i, outacc):
            tau_s = tau_v[pl.ds(b * _B + i, 16)][0]

            n = jnp.int32(0)
            for t in range(_C):
                v = dst_v[i * _K + t // 8, pl.ds((t % 8) * 16, 16)]
                mask = v <= tau_s
                plsc.store_compressed(cd_v.at[pl.ds(n, 16)], v, mask=mask)
                plsc.store_compressed(cj_v.at[pl.ds(n, 16)], iota + (t * 16),
                                      mask=mask)
                n = n + plsc.all_reduce_population_count(mask)[0]
            cd_v[pl.ds(n, 16)] = inf16
            cj_v[pl.ds(n, 16)] = bigj16

            bd = cd_v[pl.ds(0, 16)]
            bj = cj_v[pl.ds(0, 16)]
            bd, bj = plsc.sort_key_val(bd, bj)

            def merge(gi, carry):
                bd, bj = carry
                gd = cd_v[pl.ds(gi * 16, 16)]
                gj = cj_v[pl.ds(gi * 16, 16)]
                gd, gj = plsc.sort_key_val(gd, gj)
                gdr = lax.rev(gd, (0,))
                gjr = lax.rev(gj, (0,))
                take = (bd < gdr) | ((bd == gdr) & (bj < gjr))
                md = jnp.where(take, bd, gdr)
                mj = jnp.where(take, bj, gjr)
                mjs, mds = plsc.sort_key_val(mj, md)
                bd2, bj2 = plsc.sort_key_val(mds, mjs)
                return (bd2, bj2)

            nblk = (n + 15) // 16
            bd, bj = lax.fori_loop(1, nblk, merge, (bd, bj))

            rank = jnp.minimum(bj >> 7, _K - 1)
            jbase = plsc.load_gather(jb_v, [i * _K + rank])
            jcol = jbase + (bj & (_CW - 1))
            lab = plsc.load_gather(y_v, [jnp.minimum(jcol, _N - 1)])
            valid = (iota < _K) & (bd < 999.0)
            s = jnp.sum(jnp.where(valid, lab, 0.0))
            res = jnp.where(s > (_K / 2.0), 1.0, 0.0)
            return jnp.where(iota == i, res, outacc)

        acc = lax.fori_loop(0, _B, row_body, jnp.zeros((16,), jnp.float32))
        out_v[pl.ds(b * _B, 16)] = acc
        return 0

    lax.fori_loop(0, _NB, batch_body, 0)
    pltpu.sync_copy(out_v.at[pl.ds(0, _RPS)], out_hbm.at[pl.ds(row0, _RPS)])


@functools.partial(
    pl.kernel,
    out_type=jax.ShapeDtypeStruct((_N,), jnp.float32),
    mesh=plsc.VectorSubcoreMesh(core_axis_name="c", subcore_axis_name="s"),
    compiler_params=pltpu.CompilerParams(needs_layout_passes=False),
    scratch_types=[
        pltpu.VMEM((_N,), jnp.float32),
        pltpu.VMEM((_RPS * _K + 16,), jnp.int32),
        pltpu.VMEM((_RPS + 16,), jnp.float32),
        pltpu.VMEM((_B * _K,), jnp.int32),
        pltpu.VMEM((_B * _K + 16,), jnp.int32),
        pltpu.VMEM((_B * _K, _CW), jnp.float32),
        pltpu.VMEM((_CAND,), jnp.float32),
        pltpu.VMEM((_CAND,), jnp.int32),
        pltpu.VMEM((_RPS,), jnp.float32),
        pltpu.SemaphoreType.DMA,
    ],
)
def _sc_phase(d_hbm, c8_hbm, tau_hbm, y_hbm, out_hbm,
              y_v, c8_v, tau_v, idx_v, jb_v, dst_v, cd_v, cj_v, out_v, sem):
    _sc_body(d_hbm, c8_hbm, tau_hbm, y_hbm, out_hbm,
             y_v, c8_v, tau_v, idx_v, jb_v, dst_v, cd_v, cj_v, out_v, sem)


def kernel(x, y):
    n = x.shape[0]
    yf = y.astype(jnp.float32)
    d, c8, tau = _tc_phase(x)
    out = _sc_phase(
        d.reshape(n * _C, _CW),
        c8.reshape(n * _K),
        tau.reshape(n),
        yf,
    )
    return out
